# Initial kernel scaffold; baseline (speedup 1.0000x reference)
import jax
import jax.numpy as jnp
from jax.experimental import pallas as pl


def kernel(pt_likelihood, z_samples):
    raise NotImplementedError("write your pallas kernel here")


# speedup vs baseline: 2.8142x; 2.8142x over previous
import jax
import jax.numpy as jnp
from jax.experimental import pallas as pl

N_SEL = 40
NC = 1000
ROWS_PER_BLOCK = 256


def _topk_body(lik_ref, z_ref, out_ref):
    lik = lik_ref[...]
    z = z_ref[...]
    r, nc = lik.shape
    iota = jax.lax.broadcasted_iota(jnp.int32, (r, nc), 1)
    cols = []
    for _ in range(N_SEL):
        m = jnp.max(lik, axis=-1, keepdims=True)
        am = jnp.min(jnp.where(lik == m, iota, nc), axis=-1, keepdims=True)
        hit = iota == am
        zt = jnp.sum(jnp.where(hit, z, 0.0), axis=-1)
        cols.append(jnp.where(m[:, 0] == 0.0, 0.0, zt))
        lik = jnp.where(hit, -1.0, lik)
    out_ref[...] = jnp.stack(cols, axis=-1)


def kernel(pt_likelihood, z_samples):
    sb, nr, nc = pt_likelihood.shape
    rows = sb * nr
    lik = pt_likelihood.reshape(rows, nc)
    z = z_samples.reshape(rows, nc)
    r = ROWS_PER_BLOCK
    grid = (rows // r,)
    out = pl.pallas_call(
        _topk_body,
        grid=grid,
        in_specs=[
            pl.BlockSpec((r, nc), lambda i: (i, 0)),
            pl.BlockSpec((r, nc), lambda i: (i, 0)),
        ],
        out_specs=pl.BlockSpec((r, N_SEL), lambda i: (i, 0)),
        out_shape=jax.ShapeDtypeStruct((rows, N_SEL), jnp.float32),
    )(lik, z)
    return out.reshape(sb, nr, N_SEL)


# speedup vs baseline: 3.2831x; 1.1666x over previous
import jax
import jax.numpy as jnp
from jax import lax
from jax.experimental import pallas as pl
from jax.experimental.pallas import tpu as pltpu
from jax.experimental.pallas import tpu_sc as plsc

N_SEL = 40
NC = 1000
NROWS = 16384
NW = 32
ROWS_PER_W = NROWS // NW
BATCH = 16
NB = ROWS_PER_W // BATCH
NCH_FULL = 62
TSTRIDE = 65
BIG = 4096


def _sc_body(lik_hbm, z_hbm, out_hbm,
             likbuf, zbuf, tbufA, tbufB, wvA, wiA, wvB, wiB, outstage):
    w = lax.axis_index("s") * 2 + lax.axis_index("c")
    iota = lax.broadcasted_iota(jnp.int32, (16,), 0)
    base65 = iota * TSTRIDE
    neg1 = jnp.full((16,), -1.0, jnp.float32)
    zeros_f = jnp.zeros((16,), jnp.float32)
    zeros_i = jnp.zeros((16,), jnp.int32)
    lane0 = iota == 0
    tmask = iota < 8

    for off in range(0, 16 * TSTRIDE, 16):
        tbufA[pl.ds(off, 16)] = neg1
        tbufB[pl.ds(off, 16)] = neg1
    wvA[pl.ds(32, 16)] = zeros_f
    wvB[pl.ds(32, 16)] = zeros_f
    wiA[pl.ds(32, 16)] = zeros_i
    wiB[pl.ds(32, 16)] = zeros_i

    row0 = w * ROWS_PER_W

    def batch_body(b, carry):
        r0 = row0 + b * BATCH
        pltpu.sync_copy(lik_hbm.at[pl.ds(r0 * NC, BATCH * NC)],
                        likbuf.at[pl.ds(0, BATCH * NC)])
        pltpu.sync_copy(z_hbm.at[pl.ds(r0 * NC, BATCH * NC)],
                        zbuf.at[pl.ds(0, BATCH * NC)])

        def pair_body(p, carry2):
            offA = (2 * p) * NC
            offB = (2 * p + 1) * NC

            def s1(c, st):
                mA, miA, mB, miB = st
                cbase = c * 16
                idxv = iota + cbase
                vA = plsc.load_gather(likbuf, [idxv + offA])
                vB = plsc.load_gather(likbuf, [idxv + offB])
                gA = vA > mA
                mA = jnp.where(gA, vA, mA)
                miA = jnp.where(gA, idxv, miA)
                gB = vB > mB
                mB = jnp.where(gB, vB, mB)
                miB = jnp.where(gB, idxv, miB)
                plsc.store_scatter(tbufA, [base65 + c], vA)
                plsc.store_scatter(tbufB, [base65 + c], vB)
                return mA, miA, mB, miB

            st0 = (jnp.full((16,), -1.0, jnp.float32), zeros_i,
                   jnp.full((16,), -1.0, jnp.float32), zeros_i)
            mA, miA, mB, miB = lax.fori_loop(0, NCH_FULL, s1, st0)

            tidx = iota + NCH_FULL * 16
            vA = jnp.where(tmask, plsc.load_gather(likbuf, [tidx + offA]), -1.0)
            vB = jnp.where(tmask, plsc.load_gather(likbuf, [tidx + offB]), -1.0)
            gA = vA > mA
            mA = jnp.where(gA, vA, mA)
            miA = jnp.where(gA, tidx, miA)
            gB = vB > mB
            mB = jnp.where(gB, vB, mB)
            miB = jnp.where(gB, tidx, miB)
            plsc.store_scatter(tbufA, [base65 + NCH_FULL], vA)
            plsc.store_scatter(tbufB, [base65 + NCH_FULL], vB)

            def rescan(tbuf, I, m, mi):
                l = I & 15
                pos = I >> 4
                plsc.store_scatter(
                    tbuf, [jnp.full((16,), l * TSTRIDE + pos, jnp.int32)],
                    neg1, mask=lane0)
                gb = l * TSTRIDE
                v0 = plsc.load_gather(tbuf, [iota + gb])
                v1 = plsc.load_gather(tbuf, [iota + (gb + 16)])
                v2 = plsc.load_gather(tbuf, [iota + (gb + 32)])
                v3 = plsc.load_gather(tbuf, [iota + (gb + 48)])
                nm = jnp.maximum(jnp.maximum(v0, v1), jnp.maximum(v2, v3))
                NM = jnp.max(nm)
                p0 = jnp.where(v0 == NM, iota, BIG)
                p1 = jnp.where(v1 == NM, iota + 16, BIG)
                p2 = jnp.where(v2 == NM, iota + 32, BIG)
                p3 = jnp.where(v3 == NM, iota + 48, BIG)
                P = jnp.min(jnp.minimum(jnp.minimum(p0, p1),
                                        jnp.minimum(p2, p3)))
                lm = iota == l
                m = jnp.where(lm, NM, m)
                mi = jnp.where(lm, P * 16 + l, mi)
                return m, mi

            def s2(t, st):
                mA, miA, mB, miB = st
                MA = jnp.max(mA)
                IA = jnp.min(jnp.where(mA == MA, miA, BIG))
                MB = jnp.max(mB)
                IB = jnp.min(jnp.where(mB == MB, miB, BIG))
                tsplat = jnp.full((16,), t, jnp.int32)
                plsc.store_scatter(wvA, [tsplat], jnp.full((16,), MA),
                                   mask=lane0)
                plsc.store_scatter(wiA, [tsplat], jnp.full((16,), IA),
                                   mask=lane0)
                plsc.store_scatter(wvB, [tsplat], jnp.full((16,), MB),
                                   mask=lane0)
                plsc.store_scatter(wiB, [tsplat], jnp.full((16,), IB),
                                   mask=lane0)
                mA, miA = rescan(tbufA, IA, mA, miA)
                mB, miB = rescan(tbufB, IB, mB, miB)
                return mA, miA, mB, miB

            mA, miA, mB, miB = lax.fori_loop(0, N_SEL, s2,
                                             (mA, miA, mB, miB))

            for (off, wv, wi, rloc) in ((offA, wvA, wiA, 2 * p),
                                        (offB, wvB, wiB, 2 * p + 1)):
                ob = rloc * N_SEL
                for j in range(3):
                    v = wv[pl.ds(16 * j, 16)]
                    ix = wi[pl.ds(16 * j, 16)]
                    zg = plsc.load_gather(zbuf, [ix + off])
                    oz = jnp.where(v == 0.0, 0.0, zg)
                    if j < 2:
                        plsc.store_scatter(outstage, [iota + (ob + 16 * j)],
                                           oz)
                    else:
                        plsc.store_scatter(outstage, [iota + (ob + 32)],
                                           oz, mask=tmask)
            return carry2

        lax.fori_loop(0, BATCH // 2, pair_body, 0)
        pltpu.sync_copy(outstage.at[pl.ds(0, BATCH * N_SEL)],
                        out_hbm.at[pl.ds(r0 * N_SEL, BATCH * N_SEL)])
        return carry

    lax.fori_loop(0, NB, batch_body, 0)


def kernel(pt_likelihood, z_samples):
    sb, nr, nc = pt_likelihood.shape
    lik = pt_likelihood.reshape(sb * nr * nc)
    z = z_samples.reshape(sb * nr * nc)
    mesh = plsc.VectorSubcoreMesh(core_axis_name="c", subcore_axis_name="s")
    out = pl.kernel(
        _sc_body,
        out_type=jax.ShapeDtypeStruct((NROWS * N_SEL,), jnp.float32),
        mesh=mesh,
        compiler_params=pltpu.CompilerParams(needs_layout_passes=False),
        scratch_types=[
            pltpu.VMEM((BATCH * NC + 16,), jnp.float32),
            pltpu.VMEM((BATCH * NC + 16,), jnp.float32),
            pltpu.VMEM((16 * TSTRIDE,), jnp.float32),
            pltpu.VMEM((16 * TSTRIDE,), jnp.float32),
            pltpu.VMEM((48,), jnp.float32),
            pltpu.VMEM((48,), jnp.int32),
            pltpu.VMEM((48,), jnp.float32),
            pltpu.VMEM((48,), jnp.int32),
            pltpu.VMEM((BATCH * N_SEL,), jnp.float32),
        ],
    )(lik, z)
    return out.reshape(sb, nr, N_SEL)


# speedup vs baseline: 4.1799x; 1.2732x over previous
import jax
import jax.numpy as jnp
from jax import lax
from jax.experimental import pallas as pl
from jax.experimental.pallas import tpu as pltpu
from jax.experimental.pallas import tpu_sc as plsc

N_SEL = 40
NC = 1000
NROWS = 16384
NW = 32
ROWS_PER_W = NROWS // NW
BATCH = 16
NB = ROWS_PER_W // BATCH
NCH_FULL = 62
TSTRIDE = 65
BIG = 4096
R_ILV = 4


def _sc_body(lik_hbm, z_hbm, out_hbm, likbuf, zbuf, outstage, *bufs):
    tbufs = bufs[0:R_ILV]
    wvs = bufs[R_ILV:2 * R_ILV]
    wis = bufs[2 * R_ILV:3 * R_ILV]
    w = lax.axis_index("s") * 2 + lax.axis_index("c")
    iota = lax.broadcasted_iota(jnp.int32, (16,), 0)
    base65 = iota * TSTRIDE
    neg1 = jnp.full((16,), -1.0, jnp.float32)
    zeros_f = jnp.zeros((16,), jnp.float32)
    zeros_i = jnp.zeros((16,), jnp.int32)
    lane0 = iota == 0
    tmask = iota < 8

    for tb in tbufs:
        for off in range(0, 16 * TSTRIDE, 16):
            tb[pl.ds(off, 16)] = neg1
    for wv, wi in zip(wvs, wis):
        wv[pl.ds(32, 16)] = zeros_f
        wi[pl.ds(32, 16)] = zeros_i

    row0 = w * ROWS_PER_W

    def batch_body(b, carry):
        r0 = row0 + b * BATCH
        pltpu.sync_copy(lik_hbm.at[pl.ds(r0 * NC, BATCH * NC)],
                        likbuf.at[pl.ds(0, BATCH * NC)])
        pltpu.sync_copy(z_hbm.at[pl.ds(r0 * NC, BATCH * NC)],
                        zbuf.at[pl.ds(0, BATCH * NC)])

        def group_body(p, carry2):
            offs = [(R_ILV * p + i) * NC for i in range(R_ILV)]

            def s1(c, st):
                cbase = c * 16
                idxv = iota + cbase
                out = []
                for i in range(R_ILV):
                    m, mi = st[i]
                    v = plsc.load_gather(likbuf, [idxv + offs[i]])
                    g = v > m
                    out.append((jnp.where(g, v, m), jnp.where(g, idxv, mi)))
                    plsc.store_scatter(tbufs[i], [base65 + c], v)
                return tuple(out)

            st0 = tuple((jnp.full((16,), -1.0, jnp.float32), zeros_i)
                        for _ in range(R_ILV))
            st = lax.fori_loop(0, NCH_FULL, s1, st0)

            tidx = iota + NCH_FULL * 16
            st_l = []
            for i in range(R_ILV):
                m, mi = st[i]
                v = jnp.where(tmask,
                              plsc.load_gather(likbuf, [tidx + offs[i]]),
                              -1.0)
                g = v > m
                st_l.append((jnp.where(g, v, m), jnp.where(g, tidx, mi)))
                plsc.store_scatter(tbufs[i], [base65 + NCH_FULL], v)
            st = tuple(st_l)

            def rescan(tbuf, I, m, mi):
                l = I & 15
                pos = I >> 4
                plsc.store_scatter(
                    tbuf, [jnp.full((16,), l * TSTRIDE + pos, jnp.int32)],
                    neg1, mask=lane0)
                gb = l * TSTRIDE
                v0 = plsc.load_gather(tbuf, [iota + gb])
                v1 = plsc.load_gather(tbuf, [iota + (gb + 16)])
                v2 = plsc.load_gather(tbuf, [iota + (gb + 32)])
                v3 = plsc.load_gather(tbuf, [iota + (gb + 48)])
                nm = jnp.maximum(jnp.maximum(v0, v1), jnp.maximum(v2, v3))
                NM = jnp.max(nm)
                p0 = jnp.where(v0 == NM, iota, BIG)
                p1 = jnp.where(v1 == NM, iota + 16, BIG)
                p2 = jnp.where(v2 == NM, iota + 32, BIG)
                p3 = jnp.where(v3 == NM, iota + 48, BIG)
                P = jnp.min(jnp.minimum(jnp.minimum(p0, p1),
                                        jnp.minimum(p2, p3)))
                lm = iota == l
                return jnp.where(lm, NM, m), jnp.where(lm, P * 16 + l, mi)

            def s2(t, st):
                tsplat = jnp.full((16,), t, jnp.int32)
                Ms = []
                Is = []
                for i in range(R_ILV):
                    m, mi = st[i]
                    M = jnp.max(m)
                    I = jnp.min(jnp.where(m == M, mi, BIG))
                    Ms.append(M)
                    Is.append(I)
                for i in range(R_ILV):
                    plsc.store_scatter(wvs[i], [tsplat],
                                       jnp.full((16,), Ms[i]), mask=lane0)
                    plsc.store_scatter(wis[i], [tsplat],
                                       jnp.full((16,), Is[i]), mask=lane0)
                return tuple(rescan(tbufs[i], Is[i], st[i][0], st[i][1])
                             for i in range(R_ILV))

            lax.fori_loop(0, N_SEL, s2, st)

            for i in range(R_ILV):
                ob = (R_ILV * p + i) * N_SEL
                for j in range(3):
                    v = wvs[i][pl.ds(16 * j, 16)]
                    ix = wis[i][pl.ds(16 * j, 16)]
                    zg = plsc.load_gather(zbuf, [ix + offs[i]])
                    oz = jnp.where(v == 0.0, 0.0, zg)
                    if j < 2:
                        plsc.store_scatter(outstage,
                                           [iota + (ob + 16 * j)], oz)
                    else:
                        plsc.store_scatter(outstage, [iota + (ob + 32)],
                                           oz, mask=tmask)
            return carry2

        lax.fori_loop(0, BATCH // R_ILV, group_body, 0)
        pltpu.sync_copy(outstage.at[pl.ds(0, BATCH * N_SEL)],
                        out_hbm.at[pl.ds(r0 * N_SEL, BATCH * N_SEL)])
        return carry

    lax.fori_loop(0, NB, batch_body, 0)


def kernel(pt_likelihood, z_samples):
    sb, nr, nc = pt_likelihood.shape
    lik = pt_likelihood.reshape(sb * nr * nc)
    z = z_samples.reshape(sb * nr * nc)
    mesh = plsc.VectorSubcoreMesh(core_axis_name="c", subcore_axis_name="s")
    scratch = [
        pltpu.VMEM((BATCH * NC + 16,), jnp.float32),
        pltpu.VMEM((BATCH * NC + 16,), jnp.float32),
        pltpu.VMEM((BATCH * N_SEL,), jnp.float32),
    ]
    scratch += [pltpu.VMEM((16 * TSTRIDE,), jnp.float32)
                for _ in range(R_ILV)]
    scratch += [pltpu.VMEM((48,), jnp.float32) for _ in range(R_ILV)]
    scratch += [pltpu.VMEM((48,), jnp.int32) for _ in range(R_ILV)]
    out = pl.kernel(
        _sc_body,
        out_type=jax.ShapeDtypeStruct((NROWS * N_SEL,), jnp.float32),
        mesh=mesh,
        compiler_params=pltpu.CompilerParams(needs_layout_passes=False),
        scratch_types=scratch,
    )(lik, z)
    return out.reshape(sb, nr, N_SEL)
